# baseline (speedup 1.0000x reference)
import functools

import jax
import jax.numpy as jnp
from jax.experimental import pallas as pl
from jax.experimental.pallas import tpu as pltpu
from jax.experimental.pallas import tpu_sc as plsc


_LANES = 128


def _build_adjacency(idx2d, w2d, num_graphs, nodes_per_graph):
    B, V = num_graphs, nodes_per_graph
    VV = V * V
    NC, NS = 2, 16
    NW = NC * NS
    GP = B // NW
    rows_per_graph = idx2d.shape[0] // B
    zchunk = 16384

    mesh = plsc.VectorSubcoreMesh(core_axis_name="c", subcore_axis_name="s")

    @functools.partial(
        pl.kernel,
        out_type=jax.ShapeDtypeStruct((B * VV,), jnp.float32),
        mesh=mesh,
        scratch_types=[
            pltpu.VMEM((rows_per_graph, _LANES), jnp.int32),
            pltpu.VMEM((rows_per_graph, _LANES), jnp.float32),
            pltpu.VMEM((zchunk,), jnp.float32),
            pltpu.VMEM_SHARED((NS * VV,), jnp.float32),
            pltpu.SemaphoreType.DMA,
        ],
    )
    def build(idx_hbm, w_hbm, out_hbm, idx_v, w_v, zero_v, acc_sp, sem):
        c = jax.lax.axis_index("c")
        s = jax.lax.axis_index("s")
        wid = c * NS + s
        sbase = s * VV

        def zbody(i, _):
            zero_v[pl.ds(i * 16, 16)] = jnp.zeros((16,), jnp.float32)
            return 0
        jax.lax.fori_loop(0, zchunk // 16, zbody, 0)

        def one_graph(t, _):
            g = wid * GP + t
            for z in range(VV // zchunk):
                pltpu.sync_copy(zero_v, acc_sp.at[pl.ds(sbase + z * zchunk, zchunk)])
            row0 = g * rows_per_graph
            pltpu.sync_copy(idx_hbm.at[pl.ds(row0, rows_per_graph), :], idx_v)
            pltpu.sync_copy(w_hbm.at[pl.ds(row0, rows_per_graph), :], w_v)
            for j in range(rows_per_graph):
                pltpu.sync_copy(w_v.at[j], acc_sp.at[idx_v.at[j]], add=True)
            pltpu.sync_copy(acc_sp.at[pl.ds(sbase, VV)],
                            out_hbm.at[pl.ds(g * VV, VV)])
            return 0

        jax.lax.fori_loop(0, GP, one_graph, 0, unroll=True)

    return build(idx2d, w2d)


def _forward(A, emb, W1, b1, W2, b2, gw_row, gb, linW, lb, cw_row, cb,
             num_graphs, nodes_per_graph, hidden):
    B, V, H = num_graphs, nodes_per_graph, hidden
    D = emb.shape[1]
    GB = 8

    bf = jnp.bfloat16

    def body(A_ref, emb_ref, W1_ref, b1_ref, W2_ref, b2_ref, gw_ref, gb_ref,
             lw_ref, lb_ref, cw_ref, cb_ref, o1_ref, o2_ref, P_s, X_s):
        @pl.when(pl.program_id(0) == 0)
        def _():
            P_s[...] = jnp.dot(emb_ref[...].astype(bf), W1_ref[...].astype(bf),
                               preferred_element_type=jnp.float32).astype(bf)

        b1b = b1_ref[...].astype(bf)
        b2b = b2_ref[...].astype(bf)
        gwb = gw_ref[...].astype(bf)
        W2b = W2_ref[...].astype(bf)
        Pb = P_s[...]
        Abs = [A_ref[pl.ds(k * V, V), :].astype(bf) for k in range(GB)]
        H1s = [jnp.maximum(
            jnp.dot(Ab, Pb, preferred_element_type=jnp.float32).astype(bf)
            + b1b, bf(0.0)) for Ab in Abs]
        Ts = [jnp.dot(Abs[k], H1s[k],
                      preferred_element_type=jnp.float32).astype(bf)
              for k in range(GB)]
        H2s = [jnp.maximum(
            jnp.dot(t, W2b, preferred_element_type=jnp.float32).astype(bf)
            + b2b, bf(0.0)) for t in Ts]
        H2Ts = [h2.T for h2 in H2s]
        gscs = [jnp.dot(gwb, h2t, preferred_element_type=jnp.float32)
                + gb_ref[0, 0] for h2t in H2Ts]
        es = [jnp.exp(g - jnp.max(g)) for g in gscs]
        for k in range(GB):
            xs = jnp.dot(es[k].astype(bf), H2s[k],
                         preferred_element_type=jnp.float32)
            X_s[pl.ds(k, 1), :] = xs / jnp.sum(es[k])

        Xb = X_s[...].astype(bf)
        o2_ref[...] = jnp.dot(Xb, lw_ref[...].astype(bf),
                              preferred_element_type=jnp.float32) + lb_ref[...]
        z = jnp.sum(X_s[...] * cw_ref[...], axis=1, keepdims=True) + cb_ref[0, 0]
        o1_ref[...] = jnp.broadcast_to(1.0 / (1.0 + jnp.exp(-z)), (GB, 128))

    full = lambda shape: pl.BlockSpec(shape, lambda g: (0, 0))
    o1, o2 = pl.pallas_call(
        body,
        grid=(B // GB,),
        in_specs=[
            pl.BlockSpec((GB * V, V), lambda g: (g, 0)),
            full((V, D)), full((D, H)), full((1, H)),
            full((H, H)), full((1, H)),
            full((1, H)), full((1, 1)),
            full((H, H)), full((1, H)),
            full((1, H)), full((1, 1)),
        ],
        out_specs=[
            pl.BlockSpec((GB, 128), lambda g: (g, 0)),
            pl.BlockSpec((GB, H), lambda g: (g, 0)),
        ],
        out_shape=[
            jax.ShapeDtypeStruct((B, 128), jnp.float32),
            jax.ShapeDtypeStruct((B, H), jnp.float32),
        ],
        scratch_shapes=[pltpu.VMEM((V, H), jnp.bfloat16),
                        pltpu.VMEM((GB, H), jnp.float32)],
    )(A, emb, W1, b1, W2, b2, gw_row, gb, linW, lb, cw_row, cb)
    return o1, o2


def kernel(embedding, edge_weight, W1, b1, W2, b2, gateW, gateb, linW, linb,
           clsW, clsb, edge_index, node_graph_ids):
    V, D = embedding.shape
    H = W1.shape[1]
    E = edge_weight.shape[0]
    N = node_graph_ids.shape[0]
    B = N // V
    NW, NS = 32, 16
    GP = B // NW

    src = edge_index[0]
    dst = edge_index[1]
    eg = (jnp.arange(E, dtype=jnp.int32) // (E // B))
    sub = (eg // GP) % NS
    idx = sub * (V * V) + (dst % V) * V + (src % V)
    idx2d = idx.astype(jnp.int32).reshape(E // _LANES, _LANES)
    w2d = edge_weight.reshape(E // _LANES, _LANES)

    A = _build_adjacency(idx2d, w2d, B, V).reshape(B * V, V)

    o1, o2 = _forward(
        A, embedding, W1, b1.reshape(1, H), W2, b2.reshape(1, H),
        gateW.reshape(1, H), gateb.reshape(1, 1), linW, linb.reshape(1, H),
        clsW.reshape(1, H), clsb.reshape(1, 1), B, V, H)
    return o1[:, 0], o2


# speedup vs baseline: 1.0382x; 1.0382x over previous
import functools

import jax
import jax.numpy as jnp
from jax.experimental import pallas as pl
from jax.experimental.pallas import tpu as pltpu
from jax.experimental.pallas import tpu_sc as plsc


_LANES = 128


def _build_adjacency(idx2d, w2d, num_graphs, nodes_per_graph):
    B, V = num_graphs, nodes_per_graph
    VV = V * V
    NC, NS = 2, 16
    NW = NC * NS
    GP = B // NW
    rows_per_graph = idx2d.shape[0] // B
    zchunk = 16384

    mesh = plsc.VectorSubcoreMesh(core_axis_name="c", subcore_axis_name="s")

    @functools.partial(
        pl.kernel,
        out_type=jax.ShapeDtypeStruct((B * VV,), jnp.float32),
        mesh=mesh,
        scratch_types=[
            pltpu.VMEM((rows_per_graph, _LANES), jnp.int32),
            pltpu.VMEM((rows_per_graph, _LANES), jnp.float32),
            pltpu.VMEM((zchunk,), jnp.float32),
            pltpu.VMEM_SHARED((NS * VV,), jnp.float32),
            pltpu.SemaphoreType.DMA,
        ],
    )
    def build(idx_hbm, w_hbm, out_hbm, idx_v, w_v, zero_v, acc_sp, sem):
        c = jax.lax.axis_index("c")
        s = jax.lax.axis_index("s")
        wid = c * NS + s
        sbase = s * VV

        def zbody(i, _):
            zero_v[pl.ds(i * 16, 16)] = jnp.zeros((16,), jnp.float32)
            return 0
        jax.lax.fori_loop(0, zchunk // 16, zbody, 0)

        def one_graph(t, _):
            g = wid * GP + t
            for z in range(VV // zchunk):
                pltpu.sync_copy(zero_v, acc_sp.at[pl.ds(sbase + z * zchunk, zchunk)])
            row0 = g * rows_per_graph
            pltpu.sync_copy(idx_hbm.at[pl.ds(row0, rows_per_graph), :], idx_v)
            pltpu.sync_copy(w_hbm.at[pl.ds(row0, rows_per_graph), :], w_v)
            for j in range(rows_per_graph):
                pltpu.sync_copy(w_v.at[j], acc_sp.at[idx_v.at[j]], add=True)
            pltpu.sync_copy(acc_sp.at[pl.ds(sbase, VV)],
                            out_hbm.at[pl.ds(g * VV, VV)])
            return 0

        jax.lax.fori_loop(0, GP, one_graph, 0, unroll=True)

    return build(idx2d, w2d)


def _forward(A, emb, W1, b1, W2, b2, gw_row, gb, linW, lb, cw_row, cb,
             num_graphs, nodes_per_graph, hidden):
    B, V, H = num_graphs, nodes_per_graph, hidden
    D = emb.shape[1]
    GB = 8

    bf = jnp.bfloat16

    def body(A_ref, emb_ref, W1_ref, b1_ref, W2_ref, b2_ref, gw_ref, gb_ref,
             lw_ref, lb_ref, cw_ref, cb_ref, o1_ref, o2_ref, P_s, X_s):
        @pl.when(pl.program_id(0) == 0)
        def _():
            P_s[...] = jnp.dot(emb_ref[...].astype(bf), W1_ref[...].astype(bf),
                               preferred_element_type=jnp.float32).astype(bf)

        b1b = b1_ref[...].astype(bf)
        b2b = b2_ref[...].astype(bf)
        gwb = gw_ref[...].astype(bf)
        W2b = W2_ref[...].astype(bf)
        Pb = P_s[...]
        Abs = [A_ref[pl.ds(k * V, V), :].astype(bf) for k in range(GB)]
        H1s = [jnp.maximum(
            jnp.dot(Ab, Pb, preferred_element_type=jnp.float32).astype(bf)
            + b1b, bf(0.0)) for Ab in Abs]
        Ts = [jnp.dot(Abs[k], H1s[k],
                      preferred_element_type=jnp.float32).astype(bf)
              for k in range(GB)]
        H2s = [jnp.maximum(
            jnp.dot(t, W2b, preferred_element_type=jnp.float32).astype(bf)
            + b2b, bf(0.0)) for t in Ts]
        H2Ts = [h2.T for h2 in H2s]
        gscs = [jnp.dot(gwb, h2t, preferred_element_type=jnp.float32)
                + gb_ref[0, 0] for h2t in H2Ts]
        es = [jnp.exp(g - jnp.max(g)) for g in gscs]
        for k in range(GB):
            xs = jnp.dot(es[k].astype(bf), H2s[k],
                         preferred_element_type=jnp.float32)
            X_s[pl.ds(k, 1), :] = xs / jnp.sum(es[k])

        Xb = X_s[...].astype(bf)
        o2_ref[...] = jnp.dot(Xb, lw_ref[...].astype(bf),
                              preferred_element_type=jnp.float32) + lb_ref[...]
        z = jnp.sum(X_s[...] * cw_ref[...], axis=1, keepdims=True) + cb_ref[0, 0]
        o1_ref[...] = jnp.broadcast_to(1.0 / (1.0 + jnp.exp(-z)), (GB, 128))

    full = lambda shape: pl.BlockSpec(shape, lambda g: (0, 0))
    o1, o2 = pl.pallas_call(
        body,
        grid=(B // GB,),
        in_specs=[
            pl.BlockSpec((GB * V, V), lambda g: (g, 0)),
            full((V, D)), full((D, H)), full((1, H)),
            full((H, H)), full((1, H)),
            full((1, H)), full((1, 1)),
            full((H, H)), full((1, H)),
            full((1, H)), full((1, 1)),
        ],
        out_specs=[
            pl.BlockSpec((GB, 128), lambda g: (g, 0)),
            pl.BlockSpec((GB, H), lambda g: (g, 0)),
        ],
        out_shape=[
            jax.ShapeDtypeStruct((B, 128), jnp.float32),
            jax.ShapeDtypeStruct((B, H), jnp.float32),
        ],
        scratch_shapes=[pltpu.VMEM((V, H), jnp.bfloat16),
                        pltpu.VMEM((GB, H), jnp.float32)],
    )(A, emb, W1, b1, W2, b2, gw_row, gb, linW, lb, cw_row, cb)
    return o1, o2


def kernel(embedding, edge_weight, W1, b1, W2, b2, gateW, gateb, linW, linb,
           clsW, clsb, edge_index, node_graph_ids):
    V, D = embedding.shape
    H = W1.shape[1]
    E = edge_weight.shape[0]
    N = node_graph_ids.shape[0]
    B = N // V
    NW, NS = 32, 16
    GP = B // NW

    src = edge_index[0]
    dst = edge_index[1]
    HB = B // 2
    GPH = HB // NW
    eg = (jnp.arange(E, dtype=jnp.int32) // (E // B))
    sub = ((eg % HB) // GPH) % NS
    idx = sub * (V * V) + (dst % V) * V + (src % V)
    idx2d = idx.astype(jnp.int32).reshape(E // _LANES, _LANES)
    w2d = edge_weight.reshape(E // _LANES, _LANES)

    rows_half = idx2d.shape[0] // 2
    halves = []
    wargs = (embedding, W1, b1.reshape(1, H), W2, b2.reshape(1, H),
             gateW.reshape(1, H), gateb.reshape(1, 1), linW,
             linb.reshape(1, H), clsW.reshape(1, H), clsb.reshape(1, 1))
    for h in range(2):
        rows = slice(h * rows_half, (h + 1) * rows_half)
        A_h = _build_adjacency(idx2d[rows], w2d[rows], HB, V).reshape(HB * V, V)
        halves.append(_forward(A_h, *wargs, HB, V, H))
    o1 = jnp.concatenate([halves[0][0], halves[1][0]], axis=0)
    o2 = jnp.concatenate([halves[0][1], halves[1][1]], axis=0)
    return o1[:, 0], o2


# speedup vs baseline: 1.0392x; 1.0011x over previous
import functools

import jax
import jax.numpy as jnp
from jax.experimental import pallas as pl
from jax.experimental.pallas import tpu as pltpu
from jax.experimental.pallas import tpu_sc as plsc


_LANES = 128


def _build_adjacency(idx2d, w2d, num_graphs, nodes_per_graph):
    B, V = num_graphs, nodes_per_graph
    VV = V * V
    NC, NS = 2, 16
    NW = NC * NS
    GP = B // NW
    rows_per_graph = idx2d.shape[0] // B
    zchunk = 16384

    mesh = plsc.VectorSubcoreMesh(core_axis_name="c", subcore_axis_name="s")

    @functools.partial(
        pl.kernel,
        out_type=jax.ShapeDtypeStruct((B * VV,), jnp.float32),
        mesh=mesh,
        scratch_types=[
            pltpu.VMEM((rows_per_graph, _LANES), jnp.int32),
            pltpu.VMEM((rows_per_graph, _LANES), jnp.float32),
            pltpu.VMEM((zchunk,), jnp.float32),
            pltpu.VMEM_SHARED((NS * VV,), jnp.float32),
            pltpu.SemaphoreType.DMA,
        ],
    )
    def build(idx_hbm, w_hbm, out_hbm, idx_v, w_v, zero_v, acc_sp, sem):
        c = jax.lax.axis_index("c")
        s = jax.lax.axis_index("s")
        wid = c * NS + s
        sbase = s * VV

        def zbody(i, _):
            zero_v[pl.ds(i * 16, 16)] = jnp.zeros((16,), jnp.float32)
            return 0
        jax.lax.fori_loop(0, zchunk // 16, zbody, 0)

        def one_graph(t, _):
            g = wid * GP + t
            for z in range(VV // zchunk):
                pltpu.sync_copy(zero_v, acc_sp.at[pl.ds(sbase + z * zchunk, zchunk)])
            row0 = g * rows_per_graph
            pltpu.sync_copy(idx_hbm.at[pl.ds(row0, rows_per_graph), :], idx_v)
            pltpu.sync_copy(w_hbm.at[pl.ds(row0, rows_per_graph), :], w_v)
            for j in range(rows_per_graph):
                pltpu.sync_copy(w_v.at[j], acc_sp.at[idx_v.at[j]], add=True)
            pltpu.sync_copy(acc_sp.at[pl.ds(sbase, VV)],
                            out_hbm.at[pl.ds(g * VV, VV)])
            return 0

        jax.lax.fori_loop(0, GP, one_graph, 0, unroll=True)

    return build(idx2d, w2d)


def _forward(A, emb, W1, b1, W2, b2, gw_row, gb, linW, lb, cw_row, cb,
             num_graphs, nodes_per_graph, hidden):
    B, V, H = num_graphs, nodes_per_graph, hidden
    D = emb.shape[1]
    GB = 8

    bf = jnp.bfloat16

    def body(A_ref, emb_ref, W1_ref, b1_ref, W2_ref, b2_ref, gw_ref, gb_ref,
             lw_ref, lb_ref, cw_ref, cb_ref, o1_ref, o2_ref, P_s, X_s):
        @pl.when(pl.program_id(0) == 0)
        def _():
            P_s[...] = jnp.dot(emb_ref[...].astype(bf), W1_ref[...].astype(bf),
                               preferred_element_type=jnp.float32).astype(bf)

        b1b = b1_ref[...].astype(bf)
        b2b = b2_ref[...].astype(bf)
        gwb = gw_ref[...].astype(bf)
        W2b = W2_ref[...].astype(bf)
        Pb = P_s[...]
        Abs = [A_ref[pl.ds(k * V, V), :].astype(bf) for k in range(GB)]
        H1s = [jnp.maximum(
            jnp.dot(Ab, Pb, preferred_element_type=jnp.float32).astype(bf)
            + b1b, bf(0.0)) for Ab in Abs]
        Ts = [jnp.dot(Abs[k], H1s[k],
                      preferred_element_type=jnp.float32).astype(bf)
              for k in range(GB)]
        H2s = [jnp.maximum(
            jnp.dot(t, W2b, preferred_element_type=jnp.float32).astype(bf)
            + b2b, bf(0.0)) for t in Ts]
        H2Ts = [h2.T for h2 in H2s]
        gscs = [jnp.dot(gwb, h2t, preferred_element_type=jnp.float32)
                + gb_ref[0, 0] for h2t in H2Ts]
        es = [jnp.exp(g - jnp.max(g)) for g in gscs]
        for k in range(GB):
            xs = jnp.dot(es[k].astype(bf), H2s[k],
                         preferred_element_type=jnp.float32)
            X_s[pl.ds(k, 1), :] = xs / jnp.sum(es[k])

        Xb = X_s[...].astype(bf)
        o2_ref[...] = jnp.dot(Xb, lw_ref[...].astype(bf),
                              preferred_element_type=jnp.float32) + lb_ref[...]
        z = jnp.sum(X_s[...] * cw_ref[...], axis=1, keepdims=True) + cb_ref[0, 0]
        o1_ref[...] = jnp.broadcast_to(1.0 / (1.0 + jnp.exp(-z)), (GB, 128))

    full = lambda shape: pl.BlockSpec(shape, lambda g: (0, 0))
    o1, o2 = pl.pallas_call(
        body,
        grid=(B // GB,),
        in_specs=[
            pl.BlockSpec((GB * V, V), lambda g: (g, 0)),
            full((V, D)), full((D, H)), full((1, H)),
            full((H, H)), full((1, H)),
            full((1, H)), full((1, 1)),
            full((H, H)), full((1, H)),
            full((1, H)), full((1, 1)),
        ],
        out_specs=[
            pl.BlockSpec((GB, 128), lambda g: (g, 0)),
            pl.BlockSpec((GB, H), lambda g: (g, 0)),
        ],
        out_shape=[
            jax.ShapeDtypeStruct((B, 128), jnp.float32),
            jax.ShapeDtypeStruct((B, H), jnp.float32),
        ],
        scratch_shapes=[pltpu.VMEM((V, H), jnp.bfloat16),
                        pltpu.VMEM((GB, H), jnp.float32)],
    )(A, emb, W1, b1, W2, b2, gw_row, gb, linW, lb, cw_row, cb)
    return o1, o2


def kernel(embedding, edge_weight, W1, b1, W2, b2, gateW, gateb, linW, linb,
           clsW, clsb, edge_index, node_graph_ids):
    V, D = embedding.shape
    H = W1.shape[1]
    E = edge_weight.shape[0]
    N = node_graph_ids.shape[0]
    B = N // V
    NW, NS = 32, 16
    GP = B // NW

    src = edge_index[0]
    dst = edge_index[1]
    NSPLIT = 4
    HB = B // NSPLIT
    GPH = max(HB // NW, 1)
    eg = (jnp.arange(E, dtype=jnp.int32) // (E // B))
    sub = ((eg % HB) // GPH) % NS
    idx = sub * (V * V) + (dst % V) * V + (src % V)
    idx2d = idx.astype(jnp.int32).reshape(E // _LANES, _LANES)
    w2d = edge_weight.reshape(E // _LANES, _LANES)

    rows_chunk = idx2d.shape[0] // NSPLIT
    chunks = []
    wargs = (embedding, W1, b1.reshape(1, H), W2, b2.reshape(1, H),
             gateW.reshape(1, H), gateb.reshape(1, 1), linW,
             linb.reshape(1, H), clsW.reshape(1, H), clsb.reshape(1, 1))
    for h in range(NSPLIT):
        rows = slice(h * rows_chunk, (h + 1) * rows_chunk)
        A_h = _build_adjacency(idx2d[rows], w2d[rows], HB, V).reshape(HB * V, V)
        chunks.append(_forward(A_h, *wargs, HB, V, H))
    o1 = jnp.concatenate([c[0] for c in chunks], axis=0)
    o2 = jnp.concatenate([c[1] for c in chunks], axis=0)
    return o1[:, 0], o2
